# Initial kernel scaffold; baseline (speedup 1.0000x reference)
import jax
import jax.numpy as jnp
from jax.experimental import pallas as pl


def kernel(x, edge_index, W1, b1, W2, b2):
    raise NotImplementedError("write your pallas kernel here")


# speedup vs baseline: 9.1428x; 9.1428x over previous
import functools

import jax
import jax.numpy as jnp
from jax import lax
from jax.experimental import pallas as pl
from jax.experimental.pallas import tpu as pltpu
from jax.experimental.pallas import tpu_sc as plsc

_N = 10000
_E = 320000
_D = 128

_NP = 10112
_EP = 323584
_B = 128
_EPW = _EP // 32
_NCH = _EPW // _B
_STRIPE = _NP // 16

_mesh = plsc.VectorSubcoreMesh(
    core_axis_name="c", subcore_axis_name="s", num_cores=2, num_subcores=16
)


@functools.partial(
    pl.kernel,
    out_type=jax.ShapeDtypeStruct((2, _NP), jnp.float32),
    mesh=_mesh,
    scratch_types=[
        pltpu.VMEM((_B,), jnp.int32),
        pltpu.VMEM((_B,), jnp.float32),
        pltpu.VMEM_SHARED((_NP,), jnp.float32),
    ],
)
def _deg_kernel(dst_hbm, zeros_hbm, deg_out, dst_v, ones_v, deg_acc):
    c = lax.axis_index("c")
    s = lax.axis_index("s")
    wid = s * 2 + c

    @pl.when(s == 0)
    def _():
        pltpu.sync_copy(zeros_hbm, deg_acc)

    for k in range(_B // 16):
        ones_v[pl.ds(k * 16, 16)] = jnp.full((16,), 1.0, jnp.float32)
    plsc.subcore_barrier()

    def body(j, carry):
        off = wid * _EPW + j * _B
        pltpu.sync_copy(dst_hbm.at[pl.ds(off, _B)], dst_v)
        pltpu.sync_copy(ones_v, deg_acc.at[dst_v], add=True)
        return carry

    lax.fori_loop(0, _NCH, body, 0)
    plsc.subcore_barrier()

    @pl.when(s == 0)
    def _():
        pltpu.sync_copy(deg_acc, deg_out.at[c])


@functools.partial(
    pl.kernel,
    out_type=jax.ShapeDtypeStruct((2, _NP, _D), jnp.float32),
    mesh=_mesh,
    scratch_types=[
        pltpu.VMEM((_B,), jnp.int32),
        pltpu.VMEM((_B,), jnp.int32),
        pltpu.VMEM((_B, _D), jnp.float32),
        pltpu.VMEM_SHARED((_NP, _D), jnp.float32),
        pltpu.SemaphoreType.DMA,
    ],
)
def _edge_kernel(h_hbm, src_hbm, dst_hbm, zeros_hbm, acc_out,
                 src_v, dst_v, rows_v, acc, sem):
    c = lax.axis_index("c")
    s = lax.axis_index("s")
    wid = s * 2 + c

    pltpu.sync_copy(zeros_hbm, acc.at[pl.ds(s * _STRIPE, _STRIPE)])
    plsc.subcore_barrier()

    def body(j, carry):
        off = wid * _EPW + j * _B
        pltpu.sync_copy(src_hbm.at[pl.ds(off, _B)], src_v)
        pltpu.sync_copy(dst_hbm.at[pl.ds(off, _B)], dst_v)
        pltpu.async_copy(h_hbm.at[src_v], rows_v, sem).wait()
        pltpu.sync_copy(rows_v, acc.at[dst_v], add=True)
        return carry

    lax.fori_loop(0, _NCH, body, 0)
    plsc.subcore_barrier()
    pltpu.sync_copy(acc.at[pl.ds(s * _STRIPE, _STRIPE)],
                    acc_out.at[c, pl.ds(s * _STRIPE, _STRIPE)])


def _tc1_body(x_ref, w_ref, dinv_ref, o_ref):
    h = jnp.dot(x_ref[...], w_ref[...], preferred_element_type=jnp.float32,
                precision=lax.Precision.HIGHEST)
    o_ref[...] = h * dinv_ref[...]


def _tc2_body(a0_ref, a1_ref, hp_ref, dinv_ref, b_ref, w_ref, o_ref):
    t = (a0_ref[...] + a1_ref[...] + hp_ref[...]) * dinv_ref[...] + b_ref[...]
    h1 = jnp.maximum(t, 0.0)
    h2 = jnp.dot(h1, w_ref[...], preferred_element_type=jnp.float32,
                 precision=lax.Precision.HIGHEST)
    o_ref[...] = h2 * dinv_ref[...]


def _tc3_body(a0_ref, a1_ref, hp_ref, dinv_ref, b_ref, x_ref, o_ref):
    t = (a0_ref[...] + a1_ref[...] + hp_ref[...]) * dinv_ref[...] + b_ref[...]
    o_ref[...] = jnp.maximum(t, 0.0) + x_ref[...]


_row_blk = pl.BlockSpec((_D, _D), lambda i: (i, 0))
_full_w = pl.BlockSpec((_D, _D), lambda i: (0, 0))
_col_blk = pl.BlockSpec((_D, 1), lambda i: (i, 0))
_bias_blk = pl.BlockSpec((1, _D), lambda i: (0, 0))

_tc1 = pl.pallas_call(
    _tc1_body,
    grid=(_NP // _D,),
    in_specs=[_row_blk, _full_w, _col_blk],
    out_specs=_row_blk,
    out_shape=jax.ShapeDtypeStruct((_NP, _D), jnp.float32),
)

_tc2 = pl.pallas_call(
    _tc2_body,
    grid=(_NP // _D,),
    in_specs=[_row_blk, _row_blk, _row_blk, _col_blk, _bias_blk, _full_w],
    out_specs=_row_blk,
    out_shape=jax.ShapeDtypeStruct((_NP, _D), jnp.float32),
)

_tc3 = pl.pallas_call(
    _tc3_body,
    grid=(_NP // _D,),
    in_specs=[_row_blk, _row_blk, _row_blk, _col_blk, _bias_blk, _row_blk],
    out_specs=_row_blk,
    out_shape=jax.ShapeDtypeStruct((_NP, _D), jnp.float32),
)


@jax.jit
def kernel(x, edge_index, W1, b1, W2, b2):
    if x.ndim == 3:
        x = jnp.squeeze(x, axis=1)
    x_p = jnp.pad(x, ((0, _NP - _N), (0, 0)))
    pad = jnp.full((_EP - _E,), _N, jnp.int32)
    src_p = jnp.concatenate([edge_index[0], pad])
    dst_p = jnp.concatenate([edge_index[1], pad])
    zeros_n = jnp.zeros((_NP,), jnp.float32)
    zeros_blk = jnp.zeros((_STRIPE, _D), jnp.float32)

    deg2 = _deg_kernel(dst_p, zeros_n)
    dinv = lax.rsqrt(jnp.maximum(deg2[0] + deg2[1] + 1.0, 1e-12))
    dinv = dinv.reshape(_NP, 1)

    hp1 = _tc1(x_p, W1, dinv)
    acc1 = _edge_kernel(hp1, src_p, dst_p, zeros_blk)
    hp2 = _tc2(acc1[0], acc1[1], hp1, dinv, b1.reshape(1, _D), W2)
    acc2 = _edge_kernel(hp2, src_p, dst_p, zeros_blk)
    out = _tc3(acc2[0], acc2[1], hp2, dinv, b2.reshape(1, _D), x_p)
    return out[:_N]
